# Initial kernel scaffold; baseline (speedup 1.0000x reference)
import jax
import jax.numpy as jnp
from jax.experimental import pallas as pl


def kernel(x, sage1_Wl, sage1_Wr, sage1_b, sage2_Wl, sage2_Wr, sage2_b, gat_W, gat_att_src, gat_att_dst, gat_b, gcn1_W, gcn1_b, gcn2_W, gcn2_b, res1_W, res1_b, edge_index, batch):
    raise NotImplementedError("write your pallas kernel here")


# speedup vs baseline: 7.5039x; 7.5039x over previous
import functools

import jax
import jax.numpy as jnp
import numpy as np
from jax import lax
from jax.experimental import pallas as pl
from jax.experimental.pallas import tpu as pltpu
from jax.experimental.pallas import tpu_sc as plsc

_N = 10000
_NROW = 10240
_E = 160000
_H = 8
_F = 1024

_EP_S = 163840
_EP_G = 172032

_BN = 1024
_STRIPE = _NROW // 16

_f32 = jnp.float32


def _zero_fill(zbuf, d):
    z = jnp.zeros((16,), _f32)
    for r in range(16):
        for j in range(d // 16):
            zbuf[r, pl.ds(j * 16, 16)] = z


def _zero_stripe(acc, zbuf, row0):
    for t in range(_STRIPE // 16):
        pltpu.sync_copy(zbuf, acc.at[pl.ds(row0 + t * 16, 16)])


def _make_segsum(d, ch, chunks):
    nsub = ch // 128
    per_tile = ch * chunks

    @functools.partial(
        pl.kernel,
        out_type=jax.ShapeDtypeStruct((2 * _NROW, d), _f32),
        mesh=plsc.VectorSubcoreMesh(core_axis_name="c", subcore_axis_name="s"),
        compiler_params=pltpu.CompilerParams(use_tc_tiling_on_sc=False),
        scratch_types=[
            pltpu.VMEM((nsub, 128), jnp.int32),
            pltpu.VMEM((nsub, 128), jnp.int32),
            pltpu.VMEM((ch, d), _f32),
            pltpu.VMEM((16, d), _f32),
            pltpu.VMEM_SHARED((_NROW, d), _f32),
            pltpu.SemaphoreType.DMA,
            pltpu.SemaphoreType.DMA,
        ],
    )
    def k(table, sidx, didx, out, sv, dv, rows, zbuf, acc, gsem, ssem):
        c = lax.axis_index("c")
        s = lax.axis_index("s")
        _zero_fill(zbuf, d)
        row0 = s * _STRIPE
        _zero_stripe(acc, zbuf, row0)
        plsc.subcore_barrier()
        base128 = (c * 16 + s) * (per_tile // 128)

        def body(i, carry):
            off128 = base128 + i * nsub
            pltpu.sync_copy(sidx.at[pl.ds(off128, nsub)], sv)
            pltpu.sync_copy(didx.at[pl.ds(off128, nsub)], dv)
            cps = [
                pltpu.async_copy(table.at[sv.at[j]],
                                 rows.at[pl.ds(j * 128, 128)], gsem)
                for j in range(nsub)
            ]
            for cp in cps:
                cp.wait()
            cps = [
                pltpu.async_copy(rows.at[pl.ds(j * 128, 128)],
                                 acc.at[dv.at[j]], ssem, add=True)
                for j in range(nsub)
            ]
            for cp in cps:
                cp.wait()
            return carry

        lax.fori_loop(0, chunks, body, 0)
        plsc.subcore_barrier()
        pltpu.sync_copy(acc.at[pl.ds(row0, _STRIPE)],
                        out.at[pl.ds(c * _NROW + row0, _STRIPE)])

    return k


def _make_gat_edge():
    ch, chunks, nsub = 384, 14, 3
    per_tile = ch * chunks

    @functools.partial(
        pl.kernel,
        out_type=(
            jax.ShapeDtypeStruct((_EP_G, 16), _f32),
            jax.ShapeDtypeStruct((2 * _NROW, 16), _f32),
        ),
        mesh=plsc.VectorSubcoreMesh(core_axis_name="c", subcore_axis_name="s"),
        compiler_params=pltpu.CompilerParams(use_tc_tiling_on_sc=False),
        scratch_types=[
            pltpu.VMEM((nsub, 128), jnp.int32),
            pltpu.VMEM((nsub, 128), jnp.int32),
            pltpu.VMEM((ch, 16), _f32),
            pltpu.VMEM((ch, 16), _f32),
            pltpu.VMEM((16, 16), _f32),
            pltpu.VMEM((16,), _f32),
            pltpu.VMEM_SHARED((_NROW, 16), _f32),
            pltpu.SemaphoreType.DMA,
            pltpu.SemaphoreType.DMA,
        ],
    )
    def k(stab, dtab, c16, sidx, didx, ex_out, den_out,
          sv, dv, srows, drows, zbuf, cbuf, acc, gsem, ssem):
        c = lax.axis_index("c")
        s = lax.axis_index("s")
        _zero_fill(zbuf, 16)
        row0 = s * _STRIPE
        _zero_stripe(acc, zbuf, row0)
        pltpu.sync_copy(c16, cbuf)
        plsc.subcore_barrier()
        cvec = cbuf[:]
        tile = c * 16 + s
        base = tile * per_tile
        base128 = tile * (per_tile // 128)

        def body(i, carry):
            off = base + i * ch
            off128 = base128 + i * nsub
            pltpu.sync_copy(sidx.at[pl.ds(off128, nsub)], sv)
            pltpu.sync_copy(didx.at[pl.ds(off128, nsub)], dv)
            cps = [
                pltpu.async_copy(stab.at[sv.at[j]],
                                 srows.at[pl.ds(j * 128, 128)], gsem)
                for j in range(nsub)
            ] + [
                pltpu.async_copy(dtab.at[dv.at[j]],
                                 drows.at[pl.ds(j * 128, 128)], gsem)
                for j in range(nsub)
            ]
            for cp in cps:
                cp.wait()

            def ebody(e, cc):
                z = srows[e, :] + drows[e, :]
                z = jnp.maximum(z, 0.2 * z)
                srows[e, :] = jnp.exp(z - cvec)
                return cc

            lax.fori_loop(0, ch, ebody, 0)
            pltpu.sync_copy(srows, ex_out.at[pl.ds(off, ch)])
            cps = [
                pltpu.async_copy(srows.at[pl.ds(j * 128, 128)],
                                 acc.at[dv.at[j]], ssem, add=True)
                for j in range(nsub)
            ]
            for cp in cps:
                cp.wait()
            return carry

        lax.fori_loop(0, chunks, body, 0)
        plsc.subcore_barrier()
        pltpu.sync_copy(acc.at[pl.ds(row0, _STRIPE)],
                        den_out.at[pl.ds(c * _NROW + row0, _STRIPE)])

    return k


def _make_gat_agg():
    ch, nsub = 384, 3
    per_tile = _EP_G // 16
    chunks = per_tile // ch

    @functools.partial(
        pl.kernel,
        out_type=jax.ShapeDtypeStruct((16 * _NROW, 64), _f32),
        mesh=plsc.VectorSubcoreMesh(core_axis_name="c", subcore_axis_name="s"),
        compiler_params=pltpu.CompilerParams(use_tc_tiling_on_sc=False),
        scratch_types=[
            pltpu.VMEM((nsub, 128), jnp.int32),
            pltpu.VMEM((nsub, 128), jnp.int32),
            pltpu.VMEM((nsub, 128), jnp.int32),
            pltpu.VMEM((ch, 64), _f32),
            pltpu.VMEM((ch, 16), _f32),
            pltpu.VMEM((16, 64), _f32),
            pltpu.VMEM_SHARED((_NROW, 64), _f32),
            pltpu.SemaphoreType.DMA,
            pltpu.SemaphoreType.DMA,
        ],
    )
    def k(hfflat, ex, sidx, didx, out,
          sv, svh, dv, rows, exv, zbuf, acc, gsem, ssem):
        c = lax.axis_index("c")
        s = lax.axis_index("s")
        _zero_fill(zbuf, 64)
        row0 = s * _STRIPE
        base = s * per_tile
        base128 = s * (per_tile // 128)

        for hl in range(8):
            kst = hl // 2
            q = c * 8 + hl
            hoff = q * _NROW
            _zero_stripe(acc, zbuf, row0)
            plsc.subcore_barrier()

            def body(i, c2):
                off = base + i * ch
                off128 = base128 + i * nsub
                pltpu.sync_copy(sidx.at[pl.ds(off128, nsub)], sv)
                pltpu.sync_copy(didx.at[pl.ds(off128, nsub)], dv)
                pltpu.sync_copy(ex.at[pl.ds(off, ch)], exv)
                for j in range(nsub):
                    for t in range(8):
                        svh[j, pl.ds(t * 16, 16)] = (
                            sv[j, pl.ds(t * 16, 16)] + hoff)
                cps = [
                    pltpu.async_copy(hfflat.at[svh.at[j]],
                                     rows.at[pl.ds(j * 128, 128)], gsem)
                    for j in range(nsub)
                ]
                for cp in cps:
                    cp.wait()

                def ebody(e, c3):
                    exrow = exv[e, :]
                    w = jnp.broadcast_to(
                        jnp.where(c == 0, exrow[kst], exrow[kst + 4]), (16,))
                    for t in range(4):
                        rows[e, pl.ds(t * 16, 16)] = (
                            rows[e, pl.ds(t * 16, 16)] * w)
                    return c3

                lax.fori_loop(0, ch, ebody, 0)
                cps = [
                    pltpu.async_copy(rows.at[pl.ds(j * 128, 128)],
                                     acc.at[dv.at[j]], ssem, add=True)
                    for j in range(nsub)
                ]
                for cp in cps:
                    cp.wait()
                return c2

            lax.fori_loop(0, chunks, body, 0)
            plsc.subcore_barrier()
            pltpu.sync_copy(acc.at[pl.ds(row0, _STRIPE)],
                            out.at[pl.ds(hoff + row0, _STRIPE)])

    return k


_seg80 = _make_segsum(80, 512, 10)
_seg64s = _make_segsum(64, 512, 10)
_seg64g = _make_segsum(64, 384, 14)
_gat_edge = _make_gat_edge()
_gat_agg = _make_gat_agg()


def _row_spec(cols, shift=0):
    return pl.BlockSpec((_BN, cols), lambda i, _s=shift: (i + _s, 0))


def _full_spec(r, cols):
    return pl.BlockSpec((r, cols), lambda i: (0, 0))


def _tc1(xp, wcat, res_b):
    def body(x_ref, w_ref, rb_ref, t_ref, m_ref):
        y = jnp.dot(x_ref[...], w_ref[...], preferred_element_type=_f32)
        ones_col = (lax.broadcasted_iota(jnp.int32, (_BN, 16), 1) == 0
                    ).astype(_f32)
        t_ref[...] = jnp.concatenate([y[:, :64], ones_col], axis=1)
        m_ref[...] = jnp.concatenate(
            [y[:, 64:128], y[:, 128:] + rb_ref[...]], axis=1)

    return pl.pallas_call(
        body,
        grid=(10,),
        in_specs=[_row_spec(128), _full_spec(128, 192), _full_spec(1, 64)],
        out_specs=[_row_spec(80), _row_spec(128)],
        out_shape=[jax.ShapeDtypeStruct((_NROW, 80), _f32),
                   jax.ShapeDtypeStruct((_NROW, 128), _f32)],
    )(xp, wcat, res_b)


def _tc2(agg1, misc, b1):
    def body(p0, p1, m_ref, b_ref, h1_ref, ic_ref, dv_ref):
        sfull = p0[...] + p1[...]
        cnt = sfull[:, 64:65]
        invc = 1.0 / jnp.maximum(cnt, 1.0)
        mean = sfull[:, :64] * invc
        h1 = jnp.maximum(mean + m_ref[:, :64] + b_ref[...], 0.0)
        h1_ref[...] = h1 + m_ref[:, 64:]
        ic_ref[...] = invc
        dv_ref[...] = lax.rsqrt(cnt + 1.0)

    return pl.pallas_call(
        body,
        grid=(10,),
        in_specs=[_row_spec(80), _row_spec(80, 10), _row_spec(128),
                  _full_spec(1, 64)],
        out_specs=[_row_spec(64), _row_spec(1), _row_spec(1)],
        out_shape=[jax.ShapeDtypeStruct((_NROW, 64), _f32),
                   jax.ShapeDtypeStruct((_NROW, 1), _f32),
                   jax.ShapeDtypeStruct((_NROW, 1), _f32)],
    )(agg1, agg1, misc, b1)


def _tc3(agg2, h1, invc, w2cat, b2):
    def body(q0, q1, h1_ref, ic_ref, w_ref, b_ref, h2_ref):
        u = jnp.concatenate([(q0[...] + q1[...]) * ic_ref[...], h1_ref[...]],
                            axis=1)
        y = jnp.dot(u, w_ref[...], preferred_element_type=_f32)
        h2_ref[...] = jnp.maximum(y + b_ref[...], 0.0)

    return pl.pallas_call(
        body,
        grid=(10,),
        in_specs=[_row_spec(64), _row_spec(64, 10), _row_spec(64),
                  _row_spec(1), _full_spec(128, 128), _full_spec(1, 128)],
        out_specs=_row_spec(128),
        out_shape=jax.ShapeDtypeStruct((_NROW, 128), _f32),
    )(agg2, agg2, h1, invc, w2cat, b2)


def _tc4(h2, gat_W, attS, attD, bd):
    def body(h_ref, w_ref, as_ref, ad_ref, bd_ref,
             hf_ref, s_ref, d_ref, c_ref, sm):
        i = pl.program_id(0)
        hfb = jnp.dot(h_ref[...], w_ref[...], preferred_element_type=_f32)
        for q in range(16):
            hf_ref[q] = hfb[:, q * 64:(q + 1) * 64]
        a_s = jnp.dot(hfb * as_ref[...], bd_ref[...],
                      preferred_element_type=_f32)
        a_d = jnp.dot(hfb * ad_ref[...], bd_ref[...],
                      preferred_element_type=_f32)
        s_ref[...] = jnp.concatenate(
            [a_s, jnp.full((_BN, 8), -50.0, _f32)], axis=1)
        d_ref[...] = jnp.concatenate([a_d, jnp.zeros((_BN, 8), _f32)], axis=1)
        cur = jnp.concatenate(
            [jnp.max(a_s, axis=0, keepdims=True),
             jnp.max(a_d, axis=0, keepdims=True)], axis=1)

        @pl.when(i == 0)
        def _():
            sm[...] = cur

        @pl.when(i > 0)
        def _():
            sm[...] = jnp.maximum(sm[...], cur)

        @pl.when(i == 9)
        def _():
            c_ref[...] = jnp.concatenate(
                [sm[:, :8] + sm[:, 8:], jnp.zeros((1, 8), _f32)], axis=1)

    return pl.pallas_call(
        body,
        grid=(10,),
        in_specs=[_row_spec(128), _full_spec(128, _F), _full_spec(1, _F),
                  _full_spec(1, _F), _full_spec(_F, 8)],
        out_specs=[pl.BlockSpec((16, _BN, 64), lambda i: (0, i, 0)),
                   _row_spec(16), _row_spec(16), _full_spec(1, 16)],
        out_shape=[jax.ShapeDtypeStruct((16, _NROW, 64), _f32),
                   jax.ShapeDtypeStruct((_NROW, 16), _f32),
                   jax.ShapeDtypeStruct((_NROW, 16), _f32),
                   jax.ShapeDtypeStruct((1, 16), _f32)],
        scratch_shapes=[pltpu.VMEM((1, 16), _f32)],
    )(h2, gat_W, attS, attD, bd)


def _tc6(aggat, den, gat_b, gcn1_W, dinv):
    def body(*refs):
        slices = refs[:16]
        dn0, dn1, gb_ref, w_ref, dv_ref, out_ref = refs[16:]
        r = 1.0 / jnp.maximum(dn0[...] + dn1[...], 1e-16)
        g = jnp.concatenate(
            [slices[q][...] * r[:, q // 2:q // 2 + 1] for q in range(16)],
            axis=1)
        g = jnp.maximum(g + gb_ref[...], 0.0)
        out_ref[...] = jnp.dot(g, w_ref[...],
                               preferred_element_type=_f32) * dv_ref[...]

    head_specs = [
        pl.BlockSpec((_BN, 64), functools.partial(lambda q, i: (q * 10 + i, 0), q))
        for q in range(16)
    ]
    return pl.pallas_call(
        body,
        grid=(10,),
        in_specs=head_specs + [_row_spec(16), _row_spec(16, 10),
                               _full_spec(1, _F), _full_spec(_F, 64),
                               _row_spec(1)],
        out_specs=_row_spec(64),
        out_shape=jax.ShapeDtypeStruct((_NROW, 64), _f32),
    )(*([aggat] * 16), den, den, gat_b, gcn1_W, dinv)


def _tc7(r, dinv, b1):
    def body(r0, r1, dv_ref, b_ref, out_ref):
        dv = dv_ref[...]
        out_ref[...] = dv * (dv * (r0[...] + r1[...]) + b_ref[...])

    return pl.pallas_call(
        body,
        grid=(10,),
        in_specs=[_row_spec(64), _row_spec(64, 10), _row_spec(1),
                  _full_spec(1, 64)],
        out_specs=_row_spec(64),
        out_shape=jax.ShapeDtypeStruct((_NROW, 64), _f32),
    )(r, r, dinv, b1)


def _tc8(t, dinv, w2, b2):
    def body(t0, t1, dv_ref, w_ref, b_ref, out_ref):
        z = jnp.dot(t0[...] + t1[...], w_ref[...],
                    preferred_element_type=_f32)
        z = dv_ref[...] * z + b_ref[...]
        m = jnp.max(z, axis=1, keepdims=True)
        lse = jnp.log(jnp.sum(jnp.exp(z - m), axis=1, keepdims=True))
        out_ref[...] = z - m - lse

    return pl.pallas_call(
        body,
        grid=(10,),
        in_specs=[_row_spec(64), _row_spec(64, 10), _row_spec(1),
                  _full_spec(64, 40), _full_spec(1, 40)],
        out_specs=_row_spec(40),
        out_shape=jax.ShapeDtypeStruct((_NROW, 40), _f32),
    )(t, t, dinv, w2, b2)


def kernel(x, sage1_Wl, sage1_Wr, sage1_b, sage2_Wl, sage2_Wr, sage2_b,
           gat_W, gat_att_src, gat_att_dst, gat_b,
           gcn1_W, gcn1_b, gcn2_W, gcn2_b, res1_W, res1_b,
           edge_index, batch):
    src = edge_index[0].astype(jnp.int32)
    dst = edge_index[1].astype(jnp.int32)
    loop = jnp.arange(_N, dtype=jnp.int32)

    s_s = jnp.concatenate(
        [src, jnp.zeros((_EP_S - _E,), jnp.int32)]).reshape(_EP_S // 128, 128)
    d_s = jnp.concatenate(
        [dst, jnp.full((_EP_S - _E,), _N, jnp.int32)]).reshape(
            _EP_S // 128, 128)
    npad_g = _EP_G - _E - _N
    s_g = jnp.concatenate(
        [src, loop, jnp.zeros((npad_g,), jnp.int32)]).reshape(
            _EP_G // 128, 128)
    d_g = jnp.concatenate(
        [dst, loop, jnp.full((npad_g,), _N, jnp.int32)]).reshape(
            _EP_G // 128, 128)

    wcat1 = jnp.concatenate([sage1_Wl, sage1_Wr, res1_W], axis=1)
    w2cat = jnp.concatenate([sage2_Wl, sage2_Wr], axis=0)
    attS = gat_att_src.reshape(1, _F)
    attD = gat_att_dst.reshape(1, _F)
    bd = jnp.repeat(jnp.eye(_H, dtype=_f32), 128, axis=0)

    xp = jnp.pad(x, ((0, _NROW - _N), (0, 0)))
    table1, misc = _tc1(xp, wcat1, res1_b.reshape(1, 64))
    agg1 = _seg80(table1, s_s, d_s)
    h1, invc, dinv = _tc2(agg1, misc, sage1_b.reshape(1, 64))
    agg2 = _seg64s(h1, s_s, d_s)
    h2 = _tc3(agg2, h1, invc, w2cat, sage2_b.reshape(1, 128))
    hf8, stab, dtab, c16 = _tc4(h2, gat_W, attS, attD, bd)
    ex, den = _gat_edge(stab, dtab, c16.reshape(16), s_g, d_g)
    aggat = _gat_agg(hf8.reshape(16 * _NROW, 64), ex, s_g, d_g)
    tg1 = _tc6(aggat, den, gat_b.reshape(1, _F), gcn1_W, dinv)
    r = _seg64g(tg1, s_g, d_g)
    tg2 = _tc7(r, dinv, gcn1_b.reshape(1, 64))
    t2 = _seg64g(tg2, s_g, d_g)
    out = _tc8(t2, dinv, gcn2_W, gcn2_b.reshape(1, 40))
    return out[:_N]
